# baseline (speedup 1.0000x reference)
import jax
import jax.numpy as jnp
from jax import lax
from jax.experimental import pallas as pl
from jax.experimental.pallas import tpu as pltpu
from jax.experimental.pallas import tpu_sc as plsc

E = 32
D = 64
MEM = 16384
B = 4096
BT = 4864

NC = 2
NS = 16
NW = NC * NS

NBLK = 256
NSEG = MEM // NBLK
QPW = NSEG // NW
W_ROWS = MEM // NW
NJV = B // 16
GCH = 16
PACK = 16384

_sc_mesh = plsc.VectorSubcoreMesh(core_axis_name="c", subcore_axis_name="s")


def _group_body(idx_hbm, idxp_hbm, posp_hbm, packed_hbm,
                idx_all, valbuf, posbuf, offstage, vstage, pstage):
    c = lax.axis_index("c")
    s = lax.axis_index("s")
    wid = s * NC + c
    wbase = wid * W_ROWS
    pltpu.sync_copy(idx_hbm, idx_all)
    lanes = lax.iota(jnp.int32, 16)
    zeros16 = jnp.zeros((16,), jnp.int32)
    ones16 = zeros16 == zeros16

    def scan_base(j, cnt_c):
        v = idx_all[pl.ds(j * 16, 16)]
        return cnt_c + plsc.all_reduce_population_count(v < wbase)

    off_base = lax.fori_loop(0, NJV, scan_base, zeros16)
    aw = (jnp.max(off_base) + 15) // 16 * 16 + 16 * wid

    def seg(q, cnt_c):
        b0 = wbase + q * NBLK

        def scan(j, cc):
            v = idx_all[pl.ds(j * 16, 16)]
            m = (v >= b0) & (v < b0 + NBLK)
            pos = cc + plsc.cumsum(m.astype(jnp.int32)) - 1
            plsc.store_scatter(valbuf, [pos], v, mask=m)
            plsc.store_scatter(posbuf, [pos], j * 16 + lanes, mask=m)
            return cc + plsc.all_reduce_population_count(m)

        cnt_n = lax.fori_loop(0, NJV, scan, cnt_c)
        packed = (aw + jnp.max(cnt_c)) * PACK + (jnp.max(cnt_n) -
                                                 jnp.max(cnt_c))
        plsc.store_scatter(offstage, [jnp.full((16,), q, jnp.int32)],
                           jnp.full((16,), 0, jnp.int32) + packed,
                           mask=(lanes == 0))
        return cnt_n

    cnt = lax.fori_loop(0, QPW, seg, zeros16)
    pltpu.sync_copy(offstage, packed_hbm.at[pl.ds(wid * 16, 16)])

    plsc.store_scatter(valbuf, [jnp.max(cnt) + lanes], zeros16, mask=ones16)
    plsc.store_scatter(posbuf, [jnp.max(cnt) + lanes], zeros16, mask=ones16)

    nch = (jnp.max(cnt) + GCH - 1) // GCH

    def chunk(cc, carry):
        vstage[...] = valbuf[pl.ds(cc * GCH, GCH)]
        pstage[...] = posbuf[pl.ds(cc * GCH, GCH)]
        dst = aw + cc * GCH
        pltpu.sync_copy(vstage, idxp_hbm.at[pl.ds(dst, GCH)])
        pltpu.sync_copy(pstage, posp_hbm.at[pl.ds(dst, GCH)])
        return carry

    lax.fori_loop(0, nch, chunk, 0)


_group_call = pl.kernel(
    _group_body,
    out_type=(
        jax.ShapeDtypeStruct((BT,), jnp.int32),
        jax.ShapeDtypeStruct((BT,), jnp.int32),
        jax.ShapeDtypeStruct((NW * 16,), jnp.int32),
    ),
    mesh=_sc_mesh,
    compiler_params=pltpu.CompilerParams(needs_layout_passes=False),
    scratch_types=[
        pltpu.VMEM((B,), jnp.int32),
        pltpu.VMEM((B + 16,), jnp.int32),
        pltpu.VMEM((B + 16,), jnp.int32),
        pltpu.VMEM((16,), jnp.int32),
        pltpu.VMEM((GCH,), jnp.int32),
        pltpu.VMEM((GCH,), jnp.int32),
    ],
)

CH = 32


def _apply_body(packed_ref, idxp_ref, posp_ref, hid_ref, key_ref, es_ref,
                w_ref, uv_ref, o_ref, hstage, kstage, estage, ustage):
    i = pl.program_id(0)
    o_ref[...] = hid_ref[...]
    packed = packed_ref[(i // QPW) * 16 + i % QPW]
    start = packed // PACK
    n = packed % PACK
    nch = (n + CH - 1) // CH

    def chunk_body(cc, carry):
        base = start + cc * CH

        def pick(j, carry2):
            t = jnp.clip(idxp_ref[base + j] - i * NBLK, 0, NBLK - 1)
            p = jnp.clip(posp_ref[base + j], 0, B - 1)
            hstage[pl.ds(j, 1)] = hid_ref[pl.ds(t, 1)]
            kstage[pl.ds(j, 1)] = key_ref[pl.ds(t, 1)]
            estage[pl.ds(j, 1)] = es_ref[pl.ds(p, 1)]
            return carry2

        lax.fori_loop(0, CH, pick, 0)

        h3 = hstage[...]
        k3 = kstage[...]
        esg = estage[...]
        esw = jnp.dot(esg, w_ref[...], preferred_element_type=jnp.float32)
        gates = jax.nn.sigmoid(
            jnp.sum((h3 + k3) * esg[:, None, :], axis=2))
        mm = jnp.dot(h3.reshape(CH * E, D), uv_ref[...],
                     preferred_element_type=jnp.float32)
        cand = jnp.maximum(mm.reshape(CH, E, D) + esw[:, None, :], 0.0)
        ustage[...] = gates[:, :, None] * cand

        cnt = jnp.minimum(n - cc * CH, CH)

        def row(j, carry2):
            t = idxp_ref[base + j] - i * NBLK
            o_ref[pl.ds(t, 1)] = o_ref[pl.ds(t, 1)] + ustage[pl.ds(j, 1)]
            return carry2

        return lax.fori_loop(0, cnt, row, carry)

    lax.fori_loop(0, nch, chunk_body, 0)

    x = o_ref[...]
    ss = jnp.maximum(jnp.sum(x * x, axis=2, keepdims=True), 1e-12)
    o_ref[...] = x * lax.rsqrt(ss)


_apply_call = pl.pallas_call(
    _apply_body,
    grid_spec=pltpu.PrefetchScalarGridSpec(
        num_scalar_prefetch=3,
        grid=(NSEG,),
        in_specs=[
            pl.BlockSpec((NBLK, E, D), lambda i, pk, ix, ps: (i, 0, 0)),
            pl.BlockSpec((NBLK, E, D), lambda i, pk, ix, ps: (i, 0, 0)),
            pl.BlockSpec((B, D), lambda i, pk, ix, ps: (0, 0)),
            pl.BlockSpec((D, D), lambda i, pk, ix, ps: (0, 0)),
            pl.BlockSpec((D, D), lambda i, pk, ix, ps: (0, 0)),
        ],
        out_specs=pl.BlockSpec((NBLK, E, D),
                               lambda i, pk, ix, ps: (i, 0, 0)),
        scratch_shapes=[
            pltpu.VMEM((CH, E, D), jnp.float32),
            pltpu.VMEM((CH, E, D), jnp.float32),
            pltpu.VMEM((CH, D), jnp.float32),
            pltpu.VMEM((CH, E, D), jnp.float32),
        ],
    ),
    compiler_params=pltpu.CompilerParams(
        dimension_semantics=("parallel",)),
    out_shape=jax.ShapeDtypeStruct((MEM, E, D), jnp.float32),
)


def kernel(encoded_sents, indices, hiddens, keys, U, V, W):
    idxp, posp, packed = _group_call(indices)
    return _apply_call(packed, idxp, posp, hiddens, keys,
                       encoded_sents, W, U + V)


# speedup vs baseline: 1.0111x; 1.0111x over previous
import jax
import jax.numpy as jnp
from jax import lax
from jax.experimental import pallas as pl
from jax.experimental.pallas import tpu as pltpu
from jax.experimental.pallas import tpu_sc as plsc

E = 32
D = 64
MEM = 16384
B = 4096
BT = 4864

NC = 2
NS = 16
NW = NC * NS

NBLK = 256
NSEG = MEM // NBLK
QPW = NSEG // NW
W_ROWS = MEM // NW
NJV = B // 16
GCH = 16
PACK = 16384

_sc_mesh = plsc.VectorSubcoreMesh(core_axis_name="c", subcore_axis_name="s")


def _group_body(idx_hbm, idxp_hbm, posp_hbm, packed_hbm,
                idx_all, valbuf, posbuf, offstage, vstage, pstage):
    c = lax.axis_index("c")
    s = lax.axis_index("s")
    wid = s * NC + c
    wbase = wid * W_ROWS
    pltpu.sync_copy(idx_hbm, idx_all)
    lanes = lax.iota(jnp.int32, 16)
    zeros16 = jnp.zeros((16,), jnp.int32)
    ones16 = zeros16 == zeros16

    def scan_base(j, cnt_c):
        v = idx_all[pl.ds(j * 16, 16)]
        return cnt_c + plsc.all_reduce_population_count(v < wbase)

    off_base = lax.fori_loop(0, NJV, scan_base, zeros16)
    aw = (jnp.max(off_base) + 15) // 16 * 16 + 16 * wid

    def seg(q, cnt_c):
        b0 = wbase + q * NBLK

        def scan(j, cc):
            v = idx_all[pl.ds(j * 16, 16)]
            m = (v >= b0) & (v < b0 + NBLK)
            pos = cc + plsc.cumsum(m.astype(jnp.int32)) - 1
            plsc.store_scatter(valbuf, [pos], v, mask=m)
            plsc.store_scatter(posbuf, [pos], j * 16 + lanes, mask=m)
            return cc + plsc.all_reduce_population_count(m)

        cnt_n = lax.fori_loop(0, NJV, scan, cnt_c)
        packed = (aw + jnp.max(cnt_c)) * PACK + (jnp.max(cnt_n) -
                                                 jnp.max(cnt_c))
        plsc.store_scatter(offstage, [jnp.full((16,), q, jnp.int32)],
                           jnp.full((16,), 0, jnp.int32) + packed,
                           mask=(lanes == 0))
        return cnt_n

    cnt = lax.fori_loop(0, QPW, seg, zeros16)
    pltpu.sync_copy(offstage, packed_hbm.at[pl.ds(wid * 16, 16)])

    plsc.store_scatter(valbuf, [jnp.max(cnt) + lanes], zeros16, mask=ones16)
    plsc.store_scatter(posbuf, [jnp.max(cnt) + lanes], zeros16, mask=ones16)

    nch = (jnp.max(cnt) + GCH - 1) // GCH

    def chunk(cc, carry):
        vstage[...] = valbuf[pl.ds(cc * GCH, GCH)]
        pstage[...] = posbuf[pl.ds(cc * GCH, GCH)]
        dst = aw + cc * GCH
        pltpu.sync_copy(vstage, idxp_hbm.at[pl.ds(dst, GCH)])
        pltpu.sync_copy(pstage, posp_hbm.at[pl.ds(dst, GCH)])
        return carry

    lax.fori_loop(0, nch, chunk, 0)


_group_call = pl.kernel(
    _group_body,
    out_type=(
        jax.ShapeDtypeStruct((BT,), jnp.int32),
        jax.ShapeDtypeStruct((BT,), jnp.int32),
        jax.ShapeDtypeStruct((NW * 16,), jnp.int32),
    ),
    mesh=_sc_mesh,
    compiler_params=pltpu.CompilerParams(needs_layout_passes=False),
    scratch_types=[
        pltpu.VMEM((B,), jnp.int32),
        pltpu.VMEM((B + 16,), jnp.int32),
        pltpu.VMEM((B + 16,), jnp.int32),
        pltpu.VMEM((16,), jnp.int32),
        pltpu.VMEM((GCH,), jnp.int32),
        pltpu.VMEM((GCH,), jnp.int32),
    ],
)

CH = 32


def _apply_body(packed_ref, idxp_ref, posp_ref, hid_ref, key_ref, es_ref,
                w_ref, uv_ref, o_ref, hstage, kstage, estage, ustage):
    i = pl.program_id(0)
    o_ref[...] = hid_ref[...]
    packed = packed_ref[(i // QPW) * 16 + i % QPW]
    start = packed // PACK
    n = packed % PACK
    nch = (n + CH - 1) // CH

    def chunk_body(cc, carry):
        base = start + cc * CH

        for j in range(CH):
            t = jnp.clip(idxp_ref[base + j] - i * NBLK, 0, NBLK - 1)
            p = jnp.clip(posp_ref[base + j], 0, B - 1)
            hstage[pl.ds(j, 1)] = hid_ref[pl.ds(t, 1)]
            kstage[pl.ds(j, 1)] = key_ref[pl.ds(t, 1)]
            estage[pl.ds(j, 1)] = es_ref[pl.ds(p, 1)]

        h3 = hstage[...]
        k3 = kstage[...]
        esg = estage[...]
        esw = jnp.dot(esg, w_ref[...], preferred_element_type=jnp.float32)
        gates = jax.nn.sigmoid(
            jnp.sum((h3 + k3) * esg[:, None, :], axis=2))
        mm = jnp.dot(h3.reshape(CH * E, D), uv_ref[...],
                     preferred_element_type=jnp.float32)
        cand = jnp.maximum(mm.reshape(CH, E, D) + esw[:, None, :], 0.0)
        ustage[...] = gates[:, :, None] * cand

        cnt = n - cc * CH

        for j in range(CH):
            @pl.when(j < cnt)
            def _():
                t = idxp_ref[base + j] - i * NBLK
                o_ref[pl.ds(t, 1)] = (o_ref[pl.ds(t, 1)] +
                                      ustage[pl.ds(j, 1)])

        return carry

    lax.fori_loop(0, nch, chunk_body, 0)

    x = o_ref[...]
    ss = jnp.maximum(jnp.sum(x * x, axis=2, keepdims=True), 1e-12)
    o_ref[...] = x * lax.rsqrt(ss)


_apply_call = pl.pallas_call(
    _apply_body,
    grid_spec=pltpu.PrefetchScalarGridSpec(
        num_scalar_prefetch=3,
        grid=(NSEG,),
        in_specs=[
            pl.BlockSpec((NBLK, E, D), lambda i, pk, ix, ps: (i, 0, 0)),
            pl.BlockSpec((NBLK, E, D), lambda i, pk, ix, ps: (i, 0, 0)),
            pl.BlockSpec((B, D), lambda i, pk, ix, ps: (0, 0)),
            pl.BlockSpec((D, D), lambda i, pk, ix, ps: (0, 0)),
            pl.BlockSpec((D, D), lambda i, pk, ix, ps: (0, 0)),
        ],
        out_specs=pl.BlockSpec((NBLK, E, D),
                               lambda i, pk, ix, ps: (i, 0, 0)),
        scratch_shapes=[
            pltpu.VMEM((CH, E, D), jnp.float32),
            pltpu.VMEM((CH, E, D), jnp.float32),
            pltpu.VMEM((CH, D), jnp.float32),
            pltpu.VMEM((CH, E, D), jnp.float32),
        ],
    ),
    compiler_params=pltpu.CompilerParams(
        dimension_semantics=("parallel",)),
    out_shape=jax.ShapeDtypeStruct((MEM, E, D), jnp.float32),
)


def kernel(encoded_sents, indices, hiddens, keys, U, V, W):
    idxp, posp, packed = _group_call(indices)
    return _apply_call(packed, idxp, posp, hiddens, keys,
                       encoded_sents, W, U + V)
